# baseline (speedup 1.0000x reference)
import functools

import jax
import jax.numpy as jnp
from jax.experimental import pallas as pl

_M = 2048
_SLOPE = 0.01


def _lrelu(y):
    return jnp.where(y >= 0, y, _SLOPE * y)


def _scnn_dim_kernel(ll_ref, lu_ref, x_ref, w1_ref, w2_ref, w3_ref,
                     b1_ref, b2_ref, b3_ref, out_ref):
    ll = ll_ref[...].astype(jnp.bfloat16)
    lu = lu_ref[...].astype(jnp.bfloat16)

    def lap(mat, v):
        return jnp.dot(mat, v.astype(jnp.bfloat16),
                       preferred_element_type=jnp.float32)

    def mix(terms, w_ref, b_ref):
        acc = jnp.dot(terms[0], w_ref[0], preferred_element_type=jnp.float32)
        for k in range(1, len(terms)):
            acc = acc + jnp.dot(terms[k], w_ref[k],
                                preferred_element_type=jnp.float32)
        return acc + b_ref[...]

    xc = x_ref[...]
    t1 = lap(ll, xc)
    t2 = lap(lu, xc)
    t3 = lap(lu, t2)
    u = _lrelu(mix([xc, t1, t2, t3], w1_ref, b1_ref))

    u1 = lap(ll, u)
    u2 = lap(lu, u)
    u3 = lap(lu, u2)
    v = _lrelu(mix([u, u1, u2, u3], w2_ref, b2_ref))

    v1 = lap(ll, v)
    v2 = lap(lu, v)
    v3 = lap(lu, v2)
    out_ref[...] = mix([v, v1, v2, v3], w3_ref, b3_ref)


@functools.partial(jax.jit, static_argnames=())
def _run_dim(ll, lu, x, th1, b1, th2, b2, th3, b3):
    xc = x.reshape(_M, 1)
    w1 = jnp.transpose(th1, (2, 1, 0))
    w2 = jnp.transpose(th2, (2, 1, 0))
    w3 = jnp.transpose(th3, (2, 1, 0))
    b1r = b1.reshape(1, -1)
    b2r = b2.reshape(1, -1)
    b3r = b3.reshape(1, -1)
    out = pl.pallas_call(
        _scnn_dim_kernel,
        out_shape=jax.ShapeDtypeStruct((_M, 1), jnp.float32),
    )(ll, lu, xc, w1, w2, w3, b1r, b2r, b3r)
    return out.reshape(1, 1, _M)


def kernel(Ll0, Lu0, D0, adD0, x0, theta0_1, bias0_1, theta0_2, bias0_2, theta0_3, bias0_3,
           Ll1, Lu1, D1, adD1, x1, theta1_1, bias1_1, theta1_2, bias1_2, theta1_3, bias1_3,
           Ll2, Lu2, D2, adD2, x2, theta2_1, bias2_1, theta2_2, bias2_2, theta2_3, bias2_3,
           Ll3, Lu3, D3, adD3, x3, theta3_1, bias3_1, theta3_2, bias3_2, theta3_3, bias3_3,
           Ll4, Lu4, D4, adD4, x4, theta4_1, bias4_1, theta4_2, bias4_2, theta4_3, bias4_3,
           Ll5, Lu5, D5, adD5, x5, theta5_1, bias5_1, theta5_2, bias5_2, theta5_3, bias5_3):
    inp = locals()
    outs = []
    for d in range(6):
        outs.append(_run_dim(
            inp['Ll%d' % d], inp['Lu%d' % d], inp['x%d' % d],
            inp['theta%d_1' % d], inp['bias%d_1' % d],
            inp['theta%d_2' % d], inp['bias%d_2' % d],
            inp['theta%d_3' % d], inp['bias%d_3' % d]))
    return tuple(outs)


# speedup vs baseline: 1.2480x; 1.2480x over previous
import jax
import jax.numpy as jnp
from jax.experimental import pallas as pl
from jax.experimental.pallas import tpu as pltpu

_M = 2048
_NDIM = 6
_SLOPE = 0.01


def _lrelu(y):
    return jnp.where(y >= 0, y, _SLOPE * y)


def _mega_kernel(*refs):
    mats = refs[0:12]
    smalls = refs[12:12 + 7 * _NDIM]
    outs = refs[12 + 7 * _NDIM:12 + 7 * _NDIM + _NDIM]
    stage_a, stage_b, ll16, lu16, sem = refs[-5:]

    def copies(d):
        return (pltpu.make_async_copy(mats[2 * d], stage_a, sem.at[0]),
                pltpu.make_async_copy(mats[2 * d + 1], stage_b, sem.at[1]))

    for c in copies(0):
        c.start()
    for d in range(_NDIM):
        for c in copies(d):
            c.wait()
        _CH = 512
        for i in range(_M // _CH):
            sl = pl.ds(i * _CH, _CH)
            ll16[sl, :] = stage_a[sl, :].astype(jnp.bfloat16)
            lu16[sl, :] = stage_b[sl, :].astype(jnp.bfloat16)
        if d + 1 < _NDIM:
            for c in copies(d + 1):
                c.start()
        x_ref, w1_ref, w2_ref, w3_ref, b1_ref, b2_ref, b3_ref = \
            smalls[7 * d:7 * (d + 1)]

        ll = ll16[...]
        lu = lu16[...]

        def lap(mat, v):
            return jnp.dot(mat, v.astype(jnp.bfloat16),
                           preferred_element_type=jnp.float32)

        def mix(terms, w_ref, b_ref):
            acc = jnp.dot(terms[0], w_ref[0],
                          preferred_element_type=jnp.float32)
            for k in range(1, len(terms)):
                acc = acc + jnp.dot(terms[k], w_ref[k],
                                    preferred_element_type=jnp.float32)
            return acc + b_ref[...]

        xc = x_ref[...]
        t1 = lap(ll, xc)
        t2 = lap(lu, xc)
        t3 = lap(lu, t2)
        u = _lrelu(mix([xc, t1, t2, t3], w1_ref, b1_ref))

        u1 = lap(ll, u)
        u2 = lap(lu, u)
        u3 = lap(lu, u2)
        v = _lrelu(mix([u, u1, u2, u3], w2_ref, b2_ref))

        v1 = lap(ll, v)
        v2 = lap(lu, v)
        v3 = lap(lu, v2)
        outs[d][...] = mix([v, v1, v2, v3], w3_ref, b3_ref)


def kernel(Ll0, Lu0, D0, adD0, x0, theta0_1, bias0_1, theta0_2, bias0_2, theta0_3, bias0_3,
           Ll1, Lu1, D1, adD1, x1, theta1_1, bias1_1, theta1_2, bias1_2, theta1_3, bias1_3,
           Ll2, Lu2, D2, adD2, x2, theta2_1, bias2_1, theta2_2, bias2_2, theta2_3, bias2_3,
           Ll3, Lu3, D3, adD3, x3, theta3_1, bias3_1, theta3_2, bias3_2, theta3_3, bias3_3,
           Ll4, Lu4, D4, adD4, x4, theta4_1, bias4_1, theta4_2, bias4_2, theta4_3, bias4_3,
           Ll5, Lu5, D5, adD5, x5, theta5_1, bias5_1, theta5_2, bias5_2, theta5_3, bias5_3):
    inp = locals()
    mats = []
    smalls = []
    for d in range(_NDIM):
        mats += [inp['Ll%d' % d], inp['Lu%d' % d]]
        smalls += [
            inp['x%d' % d].reshape(_M, 1),
            jnp.transpose(inp['theta%d_1' % d], (2, 1, 0)),
            jnp.transpose(inp['theta%d_2' % d], (2, 1, 0)),
            jnp.transpose(inp['theta%d_3' % d], (2, 1, 0)),
            inp['bias%d_1' % d].reshape(1, -1),
            inp['bias%d_2' % d].reshape(1, -1),
            inp['bias%d_3' % d].reshape(1, -1),
        ]

    in_specs = ([pl.BlockSpec(memory_space=pl.ANY)] * len(mats)
                + [pl.BlockSpec(memory_space=pltpu.VMEM)] * len(smalls))
    outs = pl.pallas_call(
        _mega_kernel,
        out_shape=[jax.ShapeDtypeStruct((_M, 1), jnp.float32)] * _NDIM,
        in_specs=in_specs,
        out_specs=[pl.BlockSpec(memory_space=pltpu.VMEM)] * _NDIM,
        scratch_shapes=[
            pltpu.VMEM((_M, _M), jnp.float32),
            pltpu.VMEM((_M, _M), jnp.float32),
            pltpu.VMEM((_M, _M), jnp.bfloat16),
            pltpu.VMEM((_M, _M), jnp.bfloat16),
            pltpu.SemaphoreType.DMA((2,)),
        ],
        compiler_params=pltpu.CompilerParams(
            vmem_limit_bytes=128 * 1024 * 1024,
        ),
    )(*mats, *smalls)
    return tuple(o.reshape(1, 1, _M) for o in outs)


# speedup vs baseline: 1.2484x; 1.0003x over previous
import jax
import jax.numpy as jnp
from jax.experimental import pallas as pl
from jax.experimental.pallas import tpu as pltpu

_M = 2048
_NDIM = 6
_SLOPE = 0.01


def _lrelu(y):
    return jnp.where(y >= 0, y, _SLOPE * y)


def _mega_kernel(*refs):
    mats = refs[0:12]
    smalls = refs[12:12 + 7 * _NDIM]
    outs = refs[12 + 7 * _NDIM:12 + 7 * _NDIM + _NDIM]
    stage_a, stage_b, ll16, lu16, sem = refs[-5:]

    _NSPLIT = 4
    _RS = _M // _NSPLIT

    def copies(d):
        cps = []
        for j in range(_NSPLIT):
            rows = pl.ds(j * _RS, _RS)
            cps.append(pltpu.make_async_copy(
                mats[2 * d].at[rows], stage_a.at[rows], sem.at[j]))
            cps.append(pltpu.make_async_copy(
                mats[2 * d + 1].at[rows], stage_b.at[rows],
                sem.at[_NSPLIT + j]))
        return cps

    for c in copies(0):
        c.start()
    for d in range(_NDIM):
        for c in copies(d):
            c.wait()
        _CH = 512
        for i in range(_M // _CH):
            sl = pl.ds(i * _CH, _CH)
            ll16[sl, :] = stage_a[sl, :].astype(jnp.bfloat16)
            lu16[sl, :] = stage_b[sl, :].astype(jnp.bfloat16)
        if d + 1 < _NDIM:
            for c in copies(d + 1):
                c.start()
        x_ref, w1_ref, w2_ref, w3_ref, b1_ref, b2_ref, b3_ref = \
            smalls[7 * d:7 * (d + 1)]

        ll = ll16[...]
        lu = lu16[...]

        def lap(mat, v):
            return jnp.dot(mat, v.astype(jnp.bfloat16),
                           preferred_element_type=jnp.float32)

        def mix(terms, w_ref, b_ref):
            acc = jnp.dot(terms[0], w_ref[0],
                          preferred_element_type=jnp.float32)
            for k in range(1, len(terms)):
                acc = acc + jnp.dot(terms[k], w_ref[k],
                                    preferred_element_type=jnp.float32)
            return acc + b_ref[...]

        xc = x_ref[...]
        t1 = lap(ll, xc)
        t2 = lap(lu, xc)
        t3 = lap(lu, t2)
        u = _lrelu(mix([xc, t1, t2, t3], w1_ref, b1_ref))

        u1 = lap(ll, u)
        u2 = lap(lu, u)
        u3 = lap(lu, u2)
        v = _lrelu(mix([u, u1, u2, u3], w2_ref, b2_ref))

        v1 = lap(ll, v)
        v2 = lap(lu, v)
        v3 = lap(lu, v2)
        outs[d][...] = mix([v, v1, v2, v3], w3_ref, b3_ref)


def kernel(Ll0, Lu0, D0, adD0, x0, theta0_1, bias0_1, theta0_2, bias0_2, theta0_3, bias0_3,
           Ll1, Lu1, D1, adD1, x1, theta1_1, bias1_1, theta1_2, bias1_2, theta1_3, bias1_3,
           Ll2, Lu2, D2, adD2, x2, theta2_1, bias2_1, theta2_2, bias2_2, theta2_3, bias2_3,
           Ll3, Lu3, D3, adD3, x3, theta3_1, bias3_1, theta3_2, bias3_2, theta3_3, bias3_3,
           Ll4, Lu4, D4, adD4, x4, theta4_1, bias4_1, theta4_2, bias4_2, theta4_3, bias4_3,
           Ll5, Lu5, D5, adD5, x5, theta5_1, bias5_1, theta5_2, bias5_2, theta5_3, bias5_3):
    inp = locals()
    mats = []
    smalls = []
    for d in range(_NDIM):
        mats += [inp['Ll%d' % d], inp['Lu%d' % d]]
        smalls += [
            inp['x%d' % d].reshape(_M, 1),
            jnp.transpose(inp['theta%d_1' % d], (2, 1, 0)),
            jnp.transpose(inp['theta%d_2' % d], (2, 1, 0)),
            jnp.transpose(inp['theta%d_3' % d], (2, 1, 0)),
            inp['bias%d_1' % d].reshape(1, -1),
            inp['bias%d_2' % d].reshape(1, -1),
            inp['bias%d_3' % d].reshape(1, -1),
        ]

    in_specs = ([pl.BlockSpec(memory_space=pl.ANY)] * len(mats)
                + [pl.BlockSpec(memory_space=pltpu.VMEM)] * len(smalls))
    outs = pl.pallas_call(
        _mega_kernel,
        out_shape=[jax.ShapeDtypeStruct((_M, 1), jnp.float32)] * _NDIM,
        in_specs=in_specs,
        out_specs=[pl.BlockSpec(memory_space=pltpu.VMEM)] * _NDIM,
        scratch_shapes=[
            pltpu.VMEM((_M, _M), jnp.float32),
            pltpu.VMEM((_M, _M), jnp.float32),
            pltpu.VMEM((_M, _M), jnp.bfloat16),
            pltpu.VMEM((_M, _M), jnp.bfloat16),
            pltpu.SemaphoreType.DMA((8,)),
        ],
        compiler_params=pltpu.CompilerParams(
            vmem_limit_bytes=128 * 1024 * 1024,
        ),
    )(*mats, *smalls)
    return tuple(o.reshape(1, 1, _M) for o in outs)


# speedup vs baseline: 1.2926x; 1.0354x over previous
import jax
import jax.numpy as jnp
from jax.experimental import pallas as pl
from jax.experimental.pallas import tpu as pltpu

_M = 2048
_NDIM = 6
_NSPLIT = 4
_RS = _M // _NSPLIT
_SLOPE = 0.01


def _lrelu(y):
    return jnp.where(y >= 0, y, _SLOPE * y)


def _mega_kernel(*refs):
    i = pl.program_id(0)
    mats = refs[0:12]
    smalls = refs[12:12 + 7 * _NDIM]
    outs = refs[12 + 7 * _NDIM:12 + 7 * _NDIM + _NDIM]
    (stage_a, stage_b, ll16, lu16,
     xc_s, w1_s, w2_s, w3_s, b1_s, b2_s, b3_s, sem) = refs[-12:]

    def copies(d):
        cps = []
        for j in range(_NSPLIT):
            rows = pl.ds(j * _RS, _RS)
            cps.append(pltpu.make_async_copy(
                mats[2 * d].at[rows], stage_a.at[rows], sem.at[j]))
            cps.append(pltpu.make_async_copy(
                mats[2 * d + 1].at[rows], stage_b.at[rows],
                sem.at[_NSPLIT + j]))
        return cps

    @pl.when(i == 0)
    def _():
        for c in copies(0):
            c.start()

    for d in range(_NDIM):
        @pl.when(i == d)
        def _(d=d):
            for c in copies(d):
                c.wait()
            x_ref, w1_ref, w2_ref, w3_ref, b1_ref, b2_ref, b3_ref = \
                smalls[7 * d:7 * (d + 1)]
            xc_s[...] = x_ref[...]
            w1_s[...] = w1_ref[...]
            w2_s[...] = w2_ref[...]
            w3_s[...] = w3_ref[...]
            b1_s[...] = b1_ref[...]
            b2_s[...] = b2_ref[...]
            b3_s[...] = b3_ref[...]

    _CH = 512
    for c in range(_M // _CH):
        sl = pl.ds(c * _CH, _CH)
        ll16[sl, :] = stage_a[sl, :].astype(jnp.bfloat16)
        lu16[sl, :] = stage_b[sl, :].astype(jnp.bfloat16)

    for d in range(1, _NDIM):
        @pl.when(i == d - 1)
        def _(d=d):
            for c in copies(d):
                c.start()

    ll = ll16[...]
    lu = lu16[...]

    def lap(mat, v):
        return jnp.dot(mat, v.astype(jnp.bfloat16),
                       preferred_element_type=jnp.float32)

    def mix(terms, w_ref, b_ref):
        acc = jnp.dot(terms[0], w_ref[0],
                      preferred_element_type=jnp.float32)
        for k in range(1, len(terms)):
            acc = acc + jnp.dot(terms[k], w_ref[k],
                                preferred_element_type=jnp.float32)
        return acc + b_ref[...]

    xc = xc_s[...]
    t1 = lap(ll, xc)
    t2 = lap(lu, xc)
    t3 = lap(lu, t2)
    u = _lrelu(mix([xc, t1, t2, t3], w1_s, b1_s))

    u1 = lap(ll, u)
    u2 = lap(lu, u)
    u3 = lap(lu, u2)
    v = _lrelu(mix([u, u1, u2, u3], w2_s, b2_s))

    v1 = lap(ll, v)
    v2 = lap(lu, v)
    v3 = lap(lu, v2)
    y3 = None
    for k, vk in enumerate([v, v1, v2, v3]):
        term = jax.lax.dot_general(
            w3_s[k], vk, (((0,), (1,)), ((), ())),
            preferred_element_type=jnp.float32)
        y3 = term if y3 is None else y3 + term
    y3 = y3 + b3_s[...]

    for d in range(_NDIM):
        @pl.when(i == d)
        def _(d=d):
            outs[d][...] = y3


def kernel(Ll0, Lu0, D0, adD0, x0, theta0_1, bias0_1, theta0_2, bias0_2, theta0_3, bias0_3,
           Ll1, Lu1, D1, adD1, x1, theta1_1, bias1_1, theta1_2, bias1_2, theta1_3, bias1_3,
           Ll2, Lu2, D2, adD2, x2, theta2_1, bias2_1, theta2_2, bias2_2, theta2_3, bias2_3,
           Ll3, Lu3, D3, adD3, x3, theta3_1, bias3_1, theta3_2, bias3_2, theta3_3, bias3_3,
           Ll4, Lu4, D4, adD4, x4, theta4_1, bias4_1, theta4_2, bias4_2, theta4_3, bias4_3,
           Ll5, Lu5, D5, adD5, x5, theta5_1, bias5_1, theta5_2, bias5_2, theta5_3, bias5_3):
    inp = locals()
    mats = []
    smalls = []
    for d in range(_NDIM):
        mats += [inp['Ll%d' % d], inp['Lu%d' % d]]
        smalls += [
            inp['x%d' % d].reshape(_M, 1),
            jnp.transpose(inp['theta%d_1' % d], (2, 1, 0)),
            jnp.transpose(inp['theta%d_2' % d], (2, 1, 0)),
            jnp.transpose(inp['theta%d_3' % d], (2, 1, 0)),
            inp['bias%d_1' % d].reshape(1, -1),
            inp['bias%d_2' % d].reshape(1, -1),
            inp['bias%d_3' % d].reshape(1, -1),
        ]

    in_specs = ([pl.BlockSpec(memory_space=pl.ANY)] * len(mats)
                + [pl.BlockSpec(memory_space=pltpu.VMEM)] * len(smalls))
    F = smalls[1].shape[2]
    K = smalls[1].shape[0]
    outs = pl.pallas_call(
        _mega_kernel,
        grid=(_NDIM,),
        out_shape=[jax.ShapeDtypeStruct((1, _M), jnp.float32)] * _NDIM,
        in_specs=in_specs,
        out_specs=[pl.BlockSpec(memory_space=pltpu.VMEM)] * _NDIM,
        scratch_shapes=[
            pltpu.VMEM((_M, _M), jnp.float32),
            pltpu.VMEM((_M, _M), jnp.float32),
            pltpu.VMEM((_M, _M), jnp.bfloat16),
            pltpu.VMEM((_M, _M), jnp.bfloat16),
            pltpu.VMEM((_M, 1), jnp.float32),
            pltpu.VMEM((K, 1, F), jnp.float32),
            pltpu.VMEM((K, F, F), jnp.float32),
            pltpu.VMEM((K, F, 1), jnp.float32),
            pltpu.VMEM((1, F), jnp.float32),
            pltpu.VMEM((1, F), jnp.float32),
            pltpu.VMEM((1, 1), jnp.float32),
            pltpu.SemaphoreType.DMA((2 * _NSPLIT,)),
        ],
        compiler_params=pltpu.CompilerParams(
            dimension_semantics=("arbitrary",),
            vmem_limit_bytes=128 * 1024 * 1024,
        ),
    )(*mats, *smalls)
    return tuple(o.reshape(1, 1, _M) for o in outs)
